# Initial kernel scaffold; baseline (speedup 1.0000x reference)
import jax
import jax.numpy as jnp
from jax.experimental import pallas as pl


def kernel(source, source_intensity, keypoints):
    raise NotImplementedError("write your pallas kernel here")


# speedup vs baseline: 29.0272x; 29.0272x over previous
import jax
import jax.numpy as jnp
import numpy as np
from jax import lax
from jax.experimental import pallas as pl
from jax.experimental.pallas import tpu as pltpu
from jax.experimental.pallas import tpu_sc as plsc

B = 2
N = 16384
S = 1024
K = 32
NUM_CORES = 2
NUM_SUBCORES = 16
NUM_WORKERS = NUM_CORES * NUM_SUBCORES
KP_PER_WORKER = (B * S) // NUM_WORKERS
WORKERS_PER_BATCH = NUM_WORKERS // B
THR = np.float32(0.2 ** 2)


def _bf16_round(x):
    bits = plsc.bitcast(x, jnp.int32)
    lsb = jnp.bitwise_and(lax.shift_right_logical(bits, 16), 1)
    rounded = bits + (32767 + lsb)
    return plsc.bitcast(jnp.bitwise_and(rounded, jnp.int32(-65536)),
                        jnp.float32)


def _ball_query_body(src_hbm, xyzt_hbm, psq_hbm, kpt_hbm, ksq_hbm,
                     out_hbm,
                     xv, yv, zv, xbv, ybv, zbv, psqv,
                     kxv, kyv, kzv, kxbv, kybv, kzbv, ksqv,
                     idxb, idxg, rows, sem):
    wid = lax.axis_index("s") * NUM_CORES + lax.axis_index("c")
    b = wid // WORKERS_PER_BATCH
    s0 = (wid % WORKERS_PER_BATCH) * KP_PER_WORKER
    kp0 = b * S + s0

    pltpu.sync_copy(xyzt_hbm.at[pl.ds((b * 3 + 0) * N, N)], xv)
    pltpu.sync_copy(xyzt_hbm.at[pl.ds((b * 3 + 1) * N, N)], yv)
    pltpu.sync_copy(xyzt_hbm.at[pl.ds((b * 3 + 2) * N, N)], zv)
    pltpu.sync_copy(psq_hbm.at[pl.ds(b * N, N)], psqv)
    pltpu.sync_copy(kpt_hbm.at[pl.ds((b * 3 + 0) * S + s0, KP_PER_WORKER)], kxv)
    pltpu.sync_copy(kpt_hbm.at[pl.ds((b * 3 + 1) * S + s0, KP_PER_WORKER)], kyv)
    pltpu.sync_copy(kpt_hbm.at[pl.ds((b * 3 + 2) * S + s0, KP_PER_WORKER)], kzv)
    pltpu.sync_copy(ksq_hbm.at[pl.ds(b * S + s0, KP_PER_WORKER)], ksqv)

    def pre(i, carry):
        ds = pl.ds(i * 16, 16)
        xbv[ds] = _bf16_round(xv[ds])
        ybv[ds] = _bf16_round(yv[ds])
        zbv[ds] = _bf16_round(zv[ds])
        return carry

    lax.fori_loop(0, N // 16, pre, 0)

    def prek(i, carry):
        ds = pl.ds(i * 16, 16)
        kxbv[ds] = _bf16_round(kxv[ds])
        kybv[ds] = _bf16_round(kyv[ds])
        kzbv[ds] = _bf16_round(kzv[ds])
        return carry

    lax.fori_loop(0, KP_PER_WORKER // 16, prek, 0)

    lanes16 = jnp.arange(16, dtype=jnp.int32)
    zeros16 = jnp.zeros((16,), jnp.int32)

    def per_keypoint(j, _):
        jv = jnp.full((16,), j, jnp.int32)
        kx = plsc.load_gather(kxv, [jv])
        ky = plsc.load_gather(kyv, [jv])
        kz = plsc.load_gather(kzv, [jv])
        kxb = plsc.load_gather(kxbv, [jv])
        kyb = plsc.load_gather(kybv, [jv])
        kzb = plsc.load_gather(kzbv, [jv])
        ksq = plsc.load_gather(ksqv, [jv])

        def cond(carry):
            i, cntv = carry
            return jnp.logical_and(i < N // 16, jnp.all(cntv < K))

        def body(carry):
            i, cntv = carry
            xc = xbv[pl.ds(i * 16, 16)]
            yc = ybv[pl.ds(i * 16, 16)]
            zc = zbv[pl.ds(i * 16, 16)]
            psqc = psqv[pl.ds(i * 16, 16)]
            dot = (kxb * xc + kyb * yc) + kzb * zc
            d = ((-2.0) * dot + ksq) + psqc
            m = d <= THR
            c = plsc.cumsum(m.astype(jnp.int32))
            pos = cntv + c - 1
            wm = jnp.logical_and(m, pos < K)
            plsc.store_scatter(idxb, [pos], lanes16 + i * 16, mask=wm)
            return i + 1, cntv + plsc.all_reduce_population_count(m)

        _, cntv = lax.while_loop(
            cond, body, (jnp.int32(0), jnp.zeros((16,), jnp.int32)))

        first = plsc.load_gather(idxb, [zeros16])
        fillv = jnp.where(cntv == 0, jnp.full((16,), N - 1, jnp.int32), first)
        fins = []
        for j2 in range(K // 16):
            lanes = lanes16 + 16 * j2
            cur = idxb[pl.ds(16 * j2, 16)]
            fin = jnp.where(lanes < cntv, cur, fillv)
            idxg[pl.ds(16 * j2, 16)] = fin + b * N
            fins.append((lanes, fin))

        pltpu.async_copy(src_hbm.at[idxg], rows, sem).wait()
        for lanes, fin in fins:
            gx = plsc.load_gather(xv, [fin])
            gy = plsc.load_gather(yv, [fin])
            gz = plsc.load_gather(zv, [fin])
            c65 = jnp.full((16,), 65, jnp.int32)
            plsc.store_scatter(rows, [lanes, c65], gx - kx)
            plsc.store_scatter(rows, [lanes, c65 + 1], gy - ky)
            plsc.store_scatter(rows, [lanes, c65 + 2], gz - kz)
        kp = kp0 + j
        pltpu.sync_copy(rows, out_hbm.at[pl.ds(kp * K, K)])
        return _

    lax.fori_loop(0, KP_PER_WORKER, per_keypoint, 0)


@jax.jit
def kernel(source, source_intensity, keypoints):
    src_pad = jnp.concatenate(
        [source, source_intensity,
         jnp.zeros((B, N, 63), jnp.float32)], axis=-1).reshape(B * N, 128)
    xyz = source[:, :, :3]
    kpt3 = keypoints[:, :, :3]
    xyzt = jnp.transpose(xyz, (0, 2, 1)).reshape(B * 3 * N)
    kpt = jnp.transpose(kpt3, (0, 2, 1)).reshape(B * 3 * S)
    psq = jnp.sum(xyz ** 2, axis=-1).reshape(B * N)
    ksq = jnp.sum(kpt3 ** 2, axis=-1).reshape(B * S)

    mesh = plsc.VectorSubcoreMesh(
        core_axis_name="c", subcore_axis_name="s",
        num_cores=NUM_CORES, num_subcores=NUM_SUBCORES)
    call = pl.kernel(
        _ball_query_body,
        out_type=jax.ShapeDtypeStruct((B * S * K, 128), jnp.float32),
        mesh=mesh,
        compiler_params=pltpu.CompilerParams(needs_layout_passes=False),
        scratch_types=[
            pltpu.VMEM((N,), jnp.float32),
            pltpu.VMEM((N,), jnp.float32),
            pltpu.VMEM((N,), jnp.float32),
            pltpu.VMEM((N,), jnp.float32),
            pltpu.VMEM((N,), jnp.float32),
            pltpu.VMEM((N,), jnp.float32),
            pltpu.VMEM((N,), jnp.float32),
            pltpu.VMEM((KP_PER_WORKER,), jnp.float32),
            pltpu.VMEM((KP_PER_WORKER,), jnp.float32),
            pltpu.VMEM((KP_PER_WORKER,), jnp.float32),
            pltpu.VMEM((KP_PER_WORKER,), jnp.float32),
            pltpu.VMEM((KP_PER_WORKER,), jnp.float32),
            pltpu.VMEM((KP_PER_WORKER,), jnp.float32),
            pltpu.VMEM((KP_PER_WORKER,), jnp.float32),
            pltpu.VMEM((2 * K,), jnp.int32),
            pltpu.VMEM((K,), jnp.int32),
            pltpu.VMEM((K, 128), jnp.float32),
            pltpu.SemaphoreType.DMA,
        ],
    )
    out = call(src_pad, xyzt, psq, kpt, ksq)
    out = out.reshape(B, S, K, 128)
    return jnp.concatenate(
        [out[..., 65:68], out[..., 64:65], out[..., :64]], axis=-1)


# speedup vs baseline: 35.6879x; 1.2295x over previous
import jax
import jax.numpy as jnp
import numpy as np
from jax import lax
from jax.experimental import pallas as pl
from jax.experimental.pallas import tpu as pltpu
from jax.experimental.pallas import tpu_sc as plsc

B = 2
N = 16384
S = 1024
K = 32
NUM_CORES = 2
NUM_SUBCORES = 16
NUM_WORKERS = NUM_CORES * NUM_SUBCORES
KP_PER_WORKER = (B * S) // NUM_WORKERS
WORKERS_PER_BATCH = NUM_WORKERS // B
THR = np.float32(0.2 ** 2)


def _bf16_round(x):
    bits = plsc.bitcast(x, jnp.int32)
    lsb = jnp.bitwise_and(lax.shift_right_logical(bits, 16), 1)
    rounded = bits + (32767 + lsb)
    return plsc.bitcast(jnp.bitwise_and(rounded, jnp.int32(-65536)),
                        jnp.float32)


def _ball_query_body(src_hbm, xyzt_hbm, psq_hbm, kpt_hbm, ksq_hbm,
                     out_hbm,
                     xv, yv, zv, psqv, kxv, kyv, kzv, ksqv,
                     idxb, idxg0, idxg1, rows0, rows1,
                     gsem0, gsem1, osem0, osem1):
    wid = lax.axis_index("s") * NUM_CORES + lax.axis_index("c")
    b = wid // WORKERS_PER_BATCH
    s0 = (wid % WORKERS_PER_BATCH) * KP_PER_WORKER
    kp0 = b * S + s0

    pltpu.sync_copy(xyzt_hbm.at[pl.ds((b * 3 + 0) * N, N)], xv)
    pltpu.sync_copy(xyzt_hbm.at[pl.ds((b * 3 + 1) * N, N)], yv)
    pltpu.sync_copy(xyzt_hbm.at[pl.ds((b * 3 + 2) * N, N)], zv)
    pltpu.sync_copy(psq_hbm.at[pl.ds(b * N, N)], psqv)
    pltpu.sync_copy(kpt_hbm.at[pl.ds((b * 3 + 0) * S + s0, KP_PER_WORKER)], kxv)
    pltpu.sync_copy(kpt_hbm.at[pl.ds((b * 3 + 1) * S + s0, KP_PER_WORKER)], kyv)
    pltpu.sync_copy(kpt_hbm.at[pl.ds((b * 3 + 2) * S + s0, KP_PER_WORKER)], kzv)
    pltpu.sync_copy(ksq_hbm.at[pl.ds(b * S + s0, KP_PER_WORKER)], ksqv)

    lanes16 = jnp.arange(16, dtype=jnp.int32)
    zeros16 = jnp.zeros((16,), jnp.int32)

    def scan_kp(j, idxg):
        jv = jnp.full((16,), j, jnp.int32)
        kxb = _bf16_round(plsc.load_gather(kxv, [jv]))
        kyb = _bf16_round(plsc.load_gather(kyv, [jv]))
        kzb = _bf16_round(plsc.load_gather(kzv, [jv]))
        ksq = plsc.load_gather(ksqv, [jv])

        def cond(carry):
            i, cnt = carry
            return jnp.logical_and(cnt < K, i < N // 16)

        def body(carry):
            i, cnt = carry
            xc = _bf16_round(xv[pl.ds(i * 16, 16)])
            yc = _bf16_round(yv[pl.ds(i * 16, 16)])
            zc = _bf16_round(zv[pl.ds(i * 16, 16)])
            psqc = psqv[pl.ds(i * 16, 16)]
            dot = (kxb * xc + kyb * yc) + kzb * zc
            d = ((-2.0) * dot + ksq) + psqc
            m = d <= THR
            mi = m.astype(jnp.int32)
            c = plsc.cumsum(mi)
            pos = jnp.full((16,), cnt, jnp.int32) + c - 1
            wm = jnp.logical_and(m, pos < K)
            plsc.store_scatter(idxb, [pos], lanes16 + i * 16, mask=wm)
            return i + 1, cnt + jnp.sum(mi)

        _, cnt = lax.while_loop(cond, body, (jnp.int32(0), jnp.int32(0)))

        first = plsc.load_gather(idxb, [zeros16])
        cntv = jnp.full((16,), cnt, jnp.int32)
        fillv = jnp.where(cntv == 0, jnp.full((16,), N - 1, jnp.int32), first)
        for j2 in range(K // 16):
            lanes = lanes16 + 16 * j2
            cur = idxb[pl.ds(16 * j2, 16)]
            fin = jnp.where(lanes < cntv, cur, fillv)
            idxg[pl.ds(16 * j2, 16)] = fin + b * N

    def process_kp(j, idxg, rows, osem):
        jv = jnp.full((16,), j, jnp.int32)
        kx = plsc.load_gather(kxv, [jv])
        ky = plsc.load_gather(kyv, [jv])
        kz = plsc.load_gather(kzv, [jv])
        for j2 in range(K // 16):
            lanes = lanes16 + 16 * j2
            fin = idxg[pl.ds(16 * j2, 16)] - b * N
            gx = plsc.load_gather(xv, [fin])
            gy = plsc.load_gather(yv, [fin])
            gz = plsc.load_gather(zv, [fin])
            c65 = jnp.full((16,), 65, jnp.int32)
            plsc.store_scatter(rows, [lanes, c65], gx - kx)
            plsc.store_scatter(rows, [lanes, c65 + 1], gy - ky)
            plsc.store_scatter(rows, [lanes, c65 + 2], gz - kz)
        kp = kp0 + j
        pltpu.async_copy(rows, out_hbm.at[pl.ds(kp * K, K)], osem)

    def wait_gather(idxg, rows, gsem):
        pltpu.make_async_copy(src_hbm.at[idxg], rows, gsem).wait()

    def wait_out(rows, osem):
        pltpu.make_async_copy(rows, out_hbm.at[pl.ds(0, K)], osem).wait()

    def pipelined(jj, carry):
        j0 = 2 * jj
        scan_kp(j0, idxg0)

        @pl.when(jj > 0)
        def _():
            wait_gather(idxg1, rows1, gsem1)
            process_kp(j0 - 1, idxg1, rows1, osem1)
            wait_out(rows0, osem0)

        pltpu.async_copy(src_hbm.at[idxg0], rows0, gsem0)

        scan_kp(j0 + 1, idxg1)
        wait_gather(idxg0, rows0, gsem0)
        process_kp(j0, idxg0, rows0, osem0)

        @pl.when(jj > 0)
        def _():
            wait_out(rows1, osem1)

        pltpu.async_copy(src_hbm.at[idxg1], rows1, gsem1)
        return carry

    lax.fori_loop(0, KP_PER_WORKER // 2, pipelined, 0)

    wait_gather(idxg1, rows1, gsem1)
    process_kp(KP_PER_WORKER - 1, idxg1, rows1, osem1)
    wait_out(rows0, osem0)
    wait_out(rows1, osem1)


@jax.jit
def kernel(source, source_intensity, keypoints):
    src_pad = jnp.concatenate(
        [source, source_intensity,
         jnp.zeros((B, N, 63), jnp.float32)], axis=-1).reshape(B * N, 128)
    xyz = source[:, :, :3]
    kpt3 = keypoints[:, :, :3]
    xyzt = jnp.transpose(xyz, (0, 2, 1)).reshape(B * 3 * N)
    kpt = jnp.transpose(kpt3, (0, 2, 1)).reshape(B * 3 * S)
    psq = jnp.sum(xyz ** 2, axis=-1).reshape(B * N)
    ksq = jnp.sum(kpt3 ** 2, axis=-1).reshape(B * S)

    mesh = plsc.VectorSubcoreMesh(
        core_axis_name="c", subcore_axis_name="s",
        num_cores=NUM_CORES, num_subcores=NUM_SUBCORES)
    call = pl.kernel(
        _ball_query_body,
        out_type=jax.ShapeDtypeStruct((B * S * K, 128), jnp.float32),
        mesh=mesh,
        compiler_params=pltpu.CompilerParams(needs_layout_passes=False),
        scratch_types=[
            pltpu.VMEM((N,), jnp.float32),
            pltpu.VMEM((N,), jnp.float32),
            pltpu.VMEM((N,), jnp.float32),
            pltpu.VMEM((N,), jnp.float32),
            pltpu.VMEM((KP_PER_WORKER,), jnp.float32),
            pltpu.VMEM((KP_PER_WORKER,), jnp.float32),
            pltpu.VMEM((KP_PER_WORKER,), jnp.float32),
            pltpu.VMEM((KP_PER_WORKER,), jnp.float32),
            pltpu.VMEM((2 * K,), jnp.int32),
            pltpu.VMEM((K,), jnp.int32),
            pltpu.VMEM((K,), jnp.int32),
            pltpu.VMEM((K, 128), jnp.float32),
            pltpu.VMEM((K, 128), jnp.float32),
            pltpu.SemaphoreType.DMA,
            pltpu.SemaphoreType.DMA,
            pltpu.SemaphoreType.DMA,
            pltpu.SemaphoreType.DMA,
        ],
    )
    out = call(src_pad, xyzt, psq, kpt, ksq)
    out = out.reshape(B, S, K, 128)
    return jnp.concatenate(
        [out[..., 65:68], out[..., 64:65], out[..., :64]], axis=-1)


# speedup vs baseline: 52.0572x; 1.4587x over previous
import jax
import jax.numpy as jnp
import numpy as np
from jax import lax
from jax.experimental import pallas as pl
from jax.experimental.pallas import tpu as pltpu
from jax.experimental.pallas import tpu_sc as plsc

B = 2
N = 16384
S = 1024
K = 32
NUM_CORES = 2
NUM_SUBCORES = 16
NUM_WORKERS = NUM_CORES * NUM_SUBCORES
KP_PER_WORKER = (B * S) // NUM_WORKERS
WORKERS_PER_BATCH = NUM_WORKERS // B
THR = np.float32(0.2 ** 2)


def _bf16_round(x):
    bits = plsc.bitcast(x, jnp.int32)
    lsb = jnp.bitwise_and(lax.shift_right_logical(bits, 16), 1)
    rounded = bits + (32767 + lsb)
    return plsc.bitcast(jnp.bitwise_and(rounded, jnp.int32(-65536)),
                        jnp.float32)


def _ball_query_body(src_hbm, xyzt_hbm, psq_hbm, kpt_hbm, ksq_hbm,
                     out_hbm,
                     xv, yv, zv, psqv, kxv, kyv, kzv, ksqv,
                     idxb, idxg0, idxg1, rows0, rows1,
                     gsem0, gsem1, osem0, osem1):
    wid = lax.axis_index("s") * NUM_CORES + lax.axis_index("c")
    b = wid // WORKERS_PER_BATCH
    s0 = (wid % WORKERS_PER_BATCH) * KP_PER_WORKER
    kp0 = b * S + s0

    pltpu.sync_copy(xyzt_hbm.at[pl.ds((b * 3 + 0) * N, N)], xv)
    pltpu.sync_copy(xyzt_hbm.at[pl.ds((b * 3 + 1) * N, N)], yv)
    pltpu.sync_copy(xyzt_hbm.at[pl.ds((b * 3 + 2) * N, N)], zv)
    pltpu.sync_copy(psq_hbm.at[pl.ds(b * N, N)], psqv)
    pltpu.sync_copy(kpt_hbm.at[pl.ds((b * 3 + 0) * S + s0, KP_PER_WORKER)], kxv)
    pltpu.sync_copy(kpt_hbm.at[pl.ds((b * 3 + 1) * S + s0, KP_PER_WORKER)], kyv)
    pltpu.sync_copy(kpt_hbm.at[pl.ds((b * 3 + 2) * S + s0, KP_PER_WORKER)], kzv)
    pltpu.sync_copy(ksq_hbm.at[pl.ds(b * S + s0, KP_PER_WORKER)], ksqv)

    lanes16 = jnp.arange(16, dtype=jnp.int32)
    zeros16 = jnp.zeros((16,), jnp.int32)

    def scan_kp(j, idxg):
        jv = jnp.full((16,), j, jnp.int32)
        kxb = _bf16_round(plsc.load_gather(kxv, [jv]))
        kyb = _bf16_round(plsc.load_gather(kyv, [jv]))
        kzb = _bf16_round(plsc.load_gather(kzv, [jv]))
        ksq = plsc.load_gather(ksqv, [jv])

        UNROLL = 4

        def cond(carry):
            i, cntv = carry
            return jnp.logical_and(jnp.all(cntv < K), i < N // (16 * UNROLL))

        def body(carry):
            i, cntv = carry
            base = i * (16 * UNROLL)
            ms, cs, ps = [], [], []
            for u in range(UNROLL):
                ds = pl.ds(base + u * 16, 16)
                xc = _bf16_round(xv[ds])
                yc = _bf16_round(yv[ds])
                zc = _bf16_round(zv[ds])
                psqc = psqv[ds]
                dot = (kxb * xc + kyb * yc) + kzb * zc
                d = ((-2.0) * dot + ksq) + psqc
                m = d <= THR
                ms.append(m)
                cs.append(plsc.cumsum(m.astype(jnp.int32)))
                ps.append(plsc.all_reduce_population_count(m))
            off = cntv
            for u in range(UNROLL):
                pos = off + cs[u] - 1
                wm = jnp.logical_and(ms[u], pos < K)
                plsc.store_scatter(idxb, [pos],
                                   lanes16 + (base + u * 16), mask=wm)
                off = off + ps[u]
            return i + 1, off

        _, cntv = lax.while_loop(
            cond, body, (jnp.int32(0), jnp.zeros((16,), jnp.int32)))

        first = plsc.load_gather(idxb, [zeros16])
        fillv = jnp.where(cntv == 0, jnp.full((16,), N - 1, jnp.int32), first)
        for j2 in range(K // 16):
            lanes = lanes16 + 16 * j2
            cur = idxb[pl.ds(16 * j2, 16)]
            fin = jnp.where(lanes < cntv, cur, fillv)
            idxg[pl.ds(16 * j2, 16)] = fin + b * N

    def process_kp(j, idxg, rows, osem):
        jv = jnp.full((16,), j, jnp.int32)
        kx = plsc.load_gather(kxv, [jv])
        ky = plsc.load_gather(kyv, [jv])
        kz = plsc.load_gather(kzv, [jv])
        for j2 in range(K // 16):
            lanes = lanes16 + 16 * j2
            fin = idxg[pl.ds(16 * j2, 16)] - b * N
            gx = plsc.load_gather(xv, [fin])
            gy = plsc.load_gather(yv, [fin])
            gz = plsc.load_gather(zv, [fin])
            c65 = jnp.full((16,), 65, jnp.int32)
            plsc.store_scatter(rows, [lanes, c65], gx - kx)
            plsc.store_scatter(rows, [lanes, c65 + 1], gy - ky)
            plsc.store_scatter(rows, [lanes, c65 + 2], gz - kz)
        kp = kp0 + j
        pltpu.async_copy(rows, out_hbm.at[pl.ds(kp * K, K)], osem)

    def wait_gather(idxg, rows, gsem):
        pltpu.make_async_copy(src_hbm.at[idxg], rows, gsem).wait()

    def wait_out(rows, osem):
        pltpu.make_async_copy(rows, out_hbm.at[pl.ds(0, K)], osem).wait()

    def pipelined(jj, carry):
        j0 = 2 * jj
        scan_kp(j0, idxg0)

        @pl.when(jj > 0)
        def _():
            wait_gather(idxg1, rows1, gsem1)
            process_kp(j0 - 1, idxg1, rows1, osem1)
            wait_out(rows0, osem0)

        pltpu.async_copy(src_hbm.at[idxg0], rows0, gsem0)

        scan_kp(j0 + 1, idxg1)
        wait_gather(idxg0, rows0, gsem0)
        process_kp(j0, idxg0, rows0, osem0)

        @pl.when(jj > 0)
        def _():
            wait_out(rows1, osem1)

        pltpu.async_copy(src_hbm.at[idxg1], rows1, gsem1)
        return carry

    lax.fori_loop(0, KP_PER_WORKER // 2, pipelined, 0)

    wait_gather(idxg1, rows1, gsem1)
    process_kp(KP_PER_WORKER - 1, idxg1, rows1, osem1)
    wait_out(rows0, osem0)
    wait_out(rows1, osem1)


@jax.jit
def kernel(source, source_intensity, keypoints):
    src_pad = jnp.concatenate(
        [source, source_intensity,
         jnp.zeros((B, N, 63), jnp.float32)], axis=-1).reshape(B * N, 128)
    xyz = source[:, :, :3]
    kpt3 = keypoints[:, :, :3]
    xyzt = jnp.transpose(xyz, (0, 2, 1)).reshape(B * 3 * N)
    kpt = jnp.transpose(kpt3, (0, 2, 1)).reshape(B * 3 * S)
    psq = jnp.sum(xyz ** 2, axis=-1).reshape(B * N)
    ksq = jnp.sum(kpt3 ** 2, axis=-1).reshape(B * S)

    mesh = plsc.VectorSubcoreMesh(
        core_axis_name="c", subcore_axis_name="s",
        num_cores=NUM_CORES, num_subcores=NUM_SUBCORES)
    call = pl.kernel(
        _ball_query_body,
        out_type=jax.ShapeDtypeStruct((B * S * K, 128), jnp.float32),
        mesh=mesh,
        compiler_params=pltpu.CompilerParams(needs_layout_passes=False),
        scratch_types=[
            pltpu.VMEM((N,), jnp.float32),
            pltpu.VMEM((N,), jnp.float32),
            pltpu.VMEM((N,), jnp.float32),
            pltpu.VMEM((N,), jnp.float32),
            pltpu.VMEM((KP_PER_WORKER,), jnp.float32),
            pltpu.VMEM((KP_PER_WORKER,), jnp.float32),
            pltpu.VMEM((KP_PER_WORKER,), jnp.float32),
            pltpu.VMEM((KP_PER_WORKER,), jnp.float32),
            pltpu.VMEM((2 * K,), jnp.int32),
            pltpu.VMEM((K,), jnp.int32),
            pltpu.VMEM((K,), jnp.int32),
            pltpu.VMEM((K, 128), jnp.float32),
            pltpu.VMEM((K, 128), jnp.float32),
            pltpu.SemaphoreType.DMA,
            pltpu.SemaphoreType.DMA,
            pltpu.SemaphoreType.DMA,
            pltpu.SemaphoreType.DMA,
        ],
    )
    out = call(src_pad, xyzt, psq, kpt, ksq)
    out = out.reshape(B, S, K, 128)
    return jnp.concatenate(
        [out[..., 65:68], out[..., 64:65], out[..., :64]], axis=-1)


# speedup vs baseline: 54.3422x; 1.0439x over previous
import jax
import jax.numpy as jnp
import numpy as np
from jax import lax
from jax.experimental import pallas as pl
from jax.experimental.pallas import tpu as pltpu
from jax.experimental.pallas import tpu_sc as plsc

B = 2
N = 16384
S = 1024
K = 32
NUM_CORES = 2
NUM_SUBCORES = 16
NUM_WORKERS = NUM_CORES * NUM_SUBCORES
KP_PER_WORKER = (B * S) // NUM_WORKERS
WORKERS_PER_BATCH = NUM_WORKERS // B
THR = np.float32(0.2 ** 2)


def _bf16_round(x):
    bits = plsc.bitcast(x, jnp.int32)
    lsb = jnp.bitwise_and(lax.shift_right_logical(bits, 16), 1)
    rounded = bits + (32767 + lsb)
    return plsc.bitcast(jnp.bitwise_and(rounded, jnp.int32(-65536)),
                        jnp.float32)


def _ball_query_body(src_hbm, xyzt_hbm, psq_hbm, kpt_hbm, ksq_hbm,
                     out_hbm,
                     xv, yv, zv, psqv, kxv, kyv, kzv, ksqv,
                     idxb, idxg0, idxg1, rows0, rows1,
                     gsem0, gsem1, osem0, osem1):
    wid = lax.axis_index("s") * NUM_CORES + lax.axis_index("c")
    b = wid // WORKERS_PER_BATCH
    s0 = (wid % WORKERS_PER_BATCH) * KP_PER_WORKER
    kp0 = b * S + s0

    pltpu.sync_copy(xyzt_hbm.at[pl.ds((b * 3 + 0) * N, N)], xv)
    pltpu.sync_copy(xyzt_hbm.at[pl.ds((b * 3 + 1) * N, N)], yv)
    pltpu.sync_copy(xyzt_hbm.at[pl.ds((b * 3 + 2) * N, N)], zv)
    pltpu.sync_copy(psq_hbm.at[pl.ds(b * N, N)], psqv)
    pltpu.sync_copy(kpt_hbm.at[pl.ds((b * 3 + 0) * S + s0, KP_PER_WORKER)], kxv)
    pltpu.sync_copy(kpt_hbm.at[pl.ds((b * 3 + 1) * S + s0, KP_PER_WORKER)], kyv)
    pltpu.sync_copy(kpt_hbm.at[pl.ds((b * 3 + 2) * S + s0, KP_PER_WORKER)], kzv)
    pltpu.sync_copy(ksq_hbm.at[pl.ds(b * S + s0, KP_PER_WORKER)], ksqv)

    lanes16 = jnp.arange(16, dtype=jnp.int32)
    zeros16 = jnp.zeros((16,), jnp.int32)

    def scan_kp(j, idxg):
        jv = jnp.full((16,), j, jnp.int32)
        kxb = _bf16_round(plsc.load_gather(kxv, [jv]))
        kyb = _bf16_round(plsc.load_gather(kyv, [jv]))
        kzb = _bf16_round(plsc.load_gather(kzv, [jv]))
        ksq = plsc.load_gather(ksqv, [jv])

        UNROLL = 8

        def cond(carry):
            i, cntv = carry
            return jnp.logical_and(jnp.all(cntv < K), i < N // (16 * UNROLL))

        def body(carry):
            i, cntv = carry
            base = i * (16 * UNROLL)
            ms, cs, ps = [], [], []
            for u in range(UNROLL):
                ds = pl.ds(base + u * 16, 16)
                xc = _bf16_round(xv[ds])
                yc = _bf16_round(yv[ds])
                zc = _bf16_round(zv[ds])
                psqc = psqv[ds]
                dot = (kxb * xc + kyb * yc) + kzb * zc
                d = ((-2.0) * dot + ksq) + psqc
                m = d <= THR
                ms.append(m)
                cs.append(plsc.cumsum(m.astype(jnp.int32)))
                ps.append(plsc.all_reduce_population_count(m))
            off = cntv
            for u in range(UNROLL):
                pos = off + cs[u] - 1
                wm = jnp.logical_and(ms[u], pos < K)
                plsc.store_scatter(idxb, [pos],
                                   lanes16 + (base + u * 16), mask=wm)
                off = off + ps[u]
            return i + 1, off

        _, cntv = lax.while_loop(
            cond, body, (jnp.int32(0), jnp.zeros((16,), jnp.int32)))

        first = plsc.load_gather(idxb, [zeros16])
        fillv = jnp.where(cntv == 0, jnp.full((16,), N - 1, jnp.int32), first)
        for j2 in range(K // 16):
            lanes = lanes16 + 16 * j2
            cur = idxb[pl.ds(16 * j2, 16)]
            fin = jnp.where(lanes < cntv, cur, fillv)
            idxg[pl.ds(16 * j2, 16)] = fin + b * N

    def process_kp(j, idxg, rows, osem):
        jv = jnp.full((16,), j, jnp.int32)
        kx = plsc.load_gather(kxv, [jv])
        ky = plsc.load_gather(kyv, [jv])
        kz = plsc.load_gather(kzv, [jv])
        for j2 in range(K // 16):
            lanes = lanes16 + 16 * j2
            fin = idxg[pl.ds(16 * j2, 16)] - b * N
            gx = plsc.load_gather(xv, [fin])
            gy = plsc.load_gather(yv, [fin])
            gz = plsc.load_gather(zv, [fin])
            c65 = jnp.full((16,), 65, jnp.int32)
            plsc.store_scatter(rows, [lanes, c65], gx - kx)
            plsc.store_scatter(rows, [lanes, c65 + 1], gy - ky)
            plsc.store_scatter(rows, [lanes, c65 + 2], gz - kz)
        kp = kp0 + j
        pltpu.async_copy(rows, out_hbm.at[pl.ds(kp * K, K)], osem)

    def wait_gather(idxg, rows, gsem):
        pltpu.make_async_copy(src_hbm.at[idxg], rows, gsem).wait()

    def wait_out(rows, osem):
        pltpu.make_async_copy(rows, out_hbm.at[pl.ds(0, K)], osem).wait()

    def pipelined(jj, carry):
        j0 = 2 * jj
        scan_kp(j0, idxg0)

        @pl.when(jj > 0)
        def _():
            wait_gather(idxg1, rows1, gsem1)
            process_kp(j0 - 1, idxg1, rows1, osem1)
            wait_out(rows0, osem0)

        pltpu.async_copy(src_hbm.at[idxg0], rows0, gsem0)

        scan_kp(j0 + 1, idxg1)
        wait_gather(idxg0, rows0, gsem0)
        process_kp(j0, idxg0, rows0, osem0)

        @pl.when(jj > 0)
        def _():
            wait_out(rows1, osem1)

        pltpu.async_copy(src_hbm.at[idxg1], rows1, gsem1)
        return carry

    lax.fori_loop(0, KP_PER_WORKER // 2, pipelined, 0)

    wait_gather(idxg1, rows1, gsem1)
    process_kp(KP_PER_WORKER - 1, idxg1, rows1, osem1)
    wait_out(rows0, osem0)
    wait_out(rows1, osem1)


@jax.jit
def kernel(source, source_intensity, keypoints):
    src_pad = jnp.concatenate(
        [source, source_intensity,
         jnp.zeros((B, N, 63), jnp.float32)], axis=-1).reshape(B * N, 128)
    xyz = source[:, :, :3]
    kpt3 = keypoints[:, :, :3]
    xyzt = jnp.transpose(xyz, (0, 2, 1)).reshape(B * 3 * N)
    kpt = jnp.transpose(kpt3, (0, 2, 1)).reshape(B * 3 * S)
    psq = jnp.sum(xyz ** 2, axis=-1).reshape(B * N)
    ksq = jnp.sum(kpt3 ** 2, axis=-1).reshape(B * S)

    mesh = plsc.VectorSubcoreMesh(
        core_axis_name="c", subcore_axis_name="s",
        num_cores=NUM_CORES, num_subcores=NUM_SUBCORES)
    call = pl.kernel(
        _ball_query_body,
        out_type=jax.ShapeDtypeStruct((B * S * K, 128), jnp.float32),
        mesh=mesh,
        compiler_params=pltpu.CompilerParams(needs_layout_passes=False),
        scratch_types=[
            pltpu.VMEM((N,), jnp.float32),
            pltpu.VMEM((N,), jnp.float32),
            pltpu.VMEM((N,), jnp.float32),
            pltpu.VMEM((N,), jnp.float32),
            pltpu.VMEM((KP_PER_WORKER,), jnp.float32),
            pltpu.VMEM((KP_PER_WORKER,), jnp.float32),
            pltpu.VMEM((KP_PER_WORKER,), jnp.float32),
            pltpu.VMEM((KP_PER_WORKER,), jnp.float32),
            pltpu.VMEM((2 * K,), jnp.int32),
            pltpu.VMEM((K,), jnp.int32),
            pltpu.VMEM((K,), jnp.int32),
            pltpu.VMEM((K, 128), jnp.float32),
            pltpu.VMEM((K, 128), jnp.float32),
            pltpu.SemaphoreType.DMA,
            pltpu.SemaphoreType.DMA,
            pltpu.SemaphoreType.DMA,
            pltpu.SemaphoreType.DMA,
        ],
    )
    out = call(src_pad, xyzt, psq, kpt, ksq)
    out = out.reshape(B, S, K, 128)
    return jnp.concatenate(
        [out[..., 65:68], out[..., 64:65], out[..., :64]], axis=-1)


# speedup vs baseline: 84.2392x; 1.5502x over previous
import jax
import jax.numpy as jnp
import numpy as np
from jax import lax
from jax.experimental import pallas as pl
from jax.experimental.pallas import tpu as pltpu
from jax.experimental.pallas import tpu_sc as plsc

B = 2
N = 16384
S = 1024
K = 32
NUM_CORES = 2
NUM_SUBCORES = 16
NUM_WORKERS = NUM_CORES * NUM_SUBCORES
KP_PER_WORKER = (B * S) // NUM_WORKERS
WORKERS_PER_BATCH = NUM_WORKERS // B
THR = np.float32(0.2 ** 2)


def _bf16_round(x):
    bits = plsc.bitcast(x, jnp.int32)
    lsb = jnp.bitwise_and(lax.shift_right_logical(bits, 16), 1)
    rounded = bits + (32767 + lsb)
    return plsc.bitcast(jnp.bitwise_and(rounded, jnp.int32(-65536)),
                        jnp.float32)


def _ball_query_body(src_hbm, xyzt_hbm, psq_hbm, kpt_hbm, ksq_hbm,
                     out_hbm,
                     xv, yv, zv, psqv, kxv, kyv, kzv, ksqv,
                     idxb, idxg0, idxg1, rows0, rows1,
                     gsem0, gsem1, osem0, osem1):
    wid = lax.axis_index("s") * NUM_CORES + lax.axis_index("c")
    b = wid // WORKERS_PER_BATCH
    s0 = (wid % WORKERS_PER_BATCH) * KP_PER_WORKER
    kp0 = b * S + s0

    pltpu.sync_copy(xyzt_hbm.at[pl.ds((b * 3 + 0) * N, N)], xv)
    pltpu.sync_copy(xyzt_hbm.at[pl.ds((b * 3 + 1) * N, N)], yv)
    pltpu.sync_copy(xyzt_hbm.at[pl.ds((b * 3 + 2) * N, N)], zv)
    pltpu.sync_copy(psq_hbm.at[pl.ds(b * N, N)], psqv)
    pltpu.sync_copy(kpt_hbm.at[pl.ds((b * 3 + 0) * S + s0, KP_PER_WORKER)], kxv)
    pltpu.sync_copy(kpt_hbm.at[pl.ds((b * 3 + 1) * S + s0, KP_PER_WORKER)], kyv)
    pltpu.sync_copy(kpt_hbm.at[pl.ds((b * 3 + 2) * S + s0, KP_PER_WORKER)], kzv)
    pltpu.sync_copy(ksq_hbm.at[pl.ds(b * S + s0, KP_PER_WORKER)], ksqv)

    lanes16 = jnp.arange(16, dtype=jnp.int32)
    zeros16 = jnp.zeros((16,), jnp.int32)

    def scan_kp(j, idxg):
        jv = jnp.full((16,), j, jnp.int32)
        kxb = _bf16_round(plsc.load_gather(kxv, [jv]))
        kyb = _bf16_round(plsc.load_gather(kyv, [jv]))
        kzb = _bf16_round(plsc.load_gather(kzv, [jv]))
        ksq = plsc.load_gather(ksqv, [jv])

        UNROLL = 8

        def cond(carry):
            i, cntv = carry
            return jnp.logical_and(jnp.all(cntv < K), i < N // (16 * UNROLL))

        def body(carry):
            i, cntv = carry
            base = i * (16 * UNROLL)
            ms, cs, ps = [], [], []
            for u in range(UNROLL):
                ds = pl.ds(base + u * 16, 16)
                xc = _bf16_round(xv[ds])
                yc = _bf16_round(yv[ds])
                zc = _bf16_round(zv[ds])
                psqc = psqv[ds]
                dot = (kxb * xc + kyb * yc) + kzb * zc
                d = ((-2.0) * dot + ksq) + psqc
                m = d <= THR
                ms.append(m)
                cs.append(plsc.cumsum(m.astype(jnp.int32)))
                ps.append(plsc.all_reduce_population_count(m))
            off = cntv
            for u in range(UNROLL):
                pos = off + cs[u] - 1
                wm = jnp.logical_and(ms[u], pos < K)
                plsc.store_scatter(idxb, [pos],
                                   lanes16 + (base + u * 16), mask=wm)
                off = off + ps[u]
            return i + 1, off

        _, cntv = lax.while_loop(
            cond, body, (jnp.int32(0), jnp.zeros((16,), jnp.int32)))

        first = plsc.load_gather(idxb, [zeros16])
        fillv = jnp.where(cntv == 0, jnp.full((16,), N - 1, jnp.int32), first)
        for j2 in range(K // 16):
            lanes = lanes16 + 16 * j2
            cur = idxb[pl.ds(16 * j2, 16)]
            fin = jnp.where(lanes < cntv, cur, fillv)
            idxg[pl.ds(16 * j2, 16)] = fin + b * N

    def process_kp(j, idxg, rows, osem):
        jv = jnp.full((16,), j, jnp.int32)
        kx = plsc.load_gather(kxv, [jv])
        ky = plsc.load_gather(kyv, [jv])
        kz = plsc.load_gather(kzv, [jv])
        for j2 in range(K // 16):
            lanes = lanes16 + 16 * j2
            fin = idxg[pl.ds(16 * j2, 16)] - b * N
            gx = plsc.load_gather(xv, [fin])
            gy = plsc.load_gather(yv, [fin])
            gz = plsc.load_gather(zv, [fin])
            c0 = jnp.full((16,), 0, jnp.int32)
            git = plsc.load_gather(rows, [lanes, c0 + 68])
            plsc.store_scatter(rows, [lanes, c0], gx - kx)
            plsc.store_scatter(rows, [lanes, c0 + 1], gy - ky)
            plsc.store_scatter(rows, [lanes, c0 + 2], gz - kz)
            plsc.store_scatter(rows, [lanes, c0 + 3], git)
        kp = kp0 + j
        pltpu.async_copy(rows, out_hbm.at[pl.ds(kp * K, K)], osem)

    def wait_gather(idxg, rows, gsem):
        pltpu.make_async_copy(src_hbm.at[idxg], rows, gsem).wait()

    def wait_out(rows, osem):
        pltpu.make_async_copy(rows, out_hbm.at[pl.ds(0, K)], osem).wait()

    def pipelined(jj, carry):
        j0 = 2 * jj
        scan_kp(j0, idxg0)

        @pl.when(jj > 0)
        def _():
            wait_gather(idxg1, rows1, gsem1)
            process_kp(j0 - 1, idxg1, rows1, osem1)
            wait_out(rows0, osem0)

        pltpu.async_copy(src_hbm.at[idxg0], rows0, gsem0)

        scan_kp(j0 + 1, idxg1)
        wait_gather(idxg0, rows0, gsem0)
        process_kp(j0, idxg0, rows0, osem0)

        @pl.when(jj > 0)
        def _():
            wait_out(rows1, osem1)

        pltpu.async_copy(src_hbm.at[idxg1], rows1, gsem1)
        return carry

    lax.fori_loop(0, KP_PER_WORKER // 2, pipelined, 0)

    wait_gather(idxg1, rows1, gsem1)
    process_kp(KP_PER_WORKER - 1, idxg1, rows1, osem1)
    wait_out(rows0, osem0)
    wait_out(rows1, osem1)


@jax.jit
def kernel(source, source_intensity, keypoints):
    src_pad = jnp.concatenate(
        [jnp.zeros((B, N, 4), jnp.float32), source, source_intensity,
         jnp.zeros((B, N, 59), jnp.float32)], axis=-1).reshape(B * N, 128)
    xyz = source[:, :, :3]
    kpt3 = keypoints[:, :, :3]
    xyzt = jnp.transpose(xyz, (0, 2, 1)).reshape(B * 3 * N)
    kpt = jnp.transpose(kpt3, (0, 2, 1)).reshape(B * 3 * S)
    psq = jnp.sum(xyz ** 2, axis=-1).reshape(B * N)
    ksq = jnp.sum(kpt3 ** 2, axis=-1).reshape(B * S)

    mesh = plsc.VectorSubcoreMesh(
        core_axis_name="c", subcore_axis_name="s",
        num_cores=NUM_CORES, num_subcores=NUM_SUBCORES)
    call = pl.kernel(
        _ball_query_body,
        out_type=jax.ShapeDtypeStruct((B * S * K, 128), jnp.float32),
        mesh=mesh,
        compiler_params=pltpu.CompilerParams(needs_layout_passes=False),
        scratch_types=[
            pltpu.VMEM((N,), jnp.float32),
            pltpu.VMEM((N,), jnp.float32),
            pltpu.VMEM((N,), jnp.float32),
            pltpu.VMEM((N,), jnp.float32),
            pltpu.VMEM((KP_PER_WORKER,), jnp.float32),
            pltpu.VMEM((KP_PER_WORKER,), jnp.float32),
            pltpu.VMEM((KP_PER_WORKER,), jnp.float32),
            pltpu.VMEM((KP_PER_WORKER,), jnp.float32),
            pltpu.VMEM((2 * K,), jnp.int32),
            pltpu.VMEM((K,), jnp.int32),
            pltpu.VMEM((K,), jnp.int32),
            pltpu.VMEM((K, 128), jnp.float32),
            pltpu.VMEM((K, 128), jnp.float32),
            pltpu.SemaphoreType.DMA,
            pltpu.SemaphoreType.DMA,
            pltpu.SemaphoreType.DMA,
            pltpu.SemaphoreType.DMA,
        ],
    )
    out = call(src_pad, xyzt, psq, kpt, ksq)
    return out.reshape(B, S, K, 128)[..., :68]


# speedup vs baseline: 85.3812x; 1.0136x over previous
import jax
import jax.numpy as jnp
import numpy as np
from jax import lax
from jax.experimental import pallas as pl
from jax.experimental.pallas import tpu as pltpu
from jax.experimental.pallas import tpu_sc as plsc

B = 2
N = 16384
S = 1024
K = 32
NUM_CORES = 2
NUM_SUBCORES = 16
NUM_WORKERS = NUM_CORES * NUM_SUBCORES
KP_PER_WORKER = (B * S) // NUM_WORKERS
WORKERS_PER_BATCH = NUM_WORKERS // B
THR = np.float32(0.2 ** 2)


def _bf16_round(x):
    bits = plsc.bitcast(x, jnp.int32)
    lsb = jnp.bitwise_and(lax.shift_right_logical(bits, 16), 1)
    rounded = bits + (32767 + lsb)
    return plsc.bitcast(jnp.bitwise_and(rounded, jnp.int32(-65536)),
                        jnp.float32)


def _ball_query_body(src_hbm, xyzt_hbm, psq_hbm, kpt_hbm, ksq_hbm,
                     out_hbm,
                     xv, yv, zv, psqv, kxv, kyv, kzv, ksqv,
                     idxb, idxg0, idxg1, rows0, rows1,
                     gsem0, gsem1, osem0, osem1):
    wid = lax.axis_index("s") * NUM_CORES + lax.axis_index("c")
    b = wid // WORKERS_PER_BATCH
    s0 = (wid % WORKERS_PER_BATCH) * KP_PER_WORKER
    kp0 = b * S + s0

    pltpu.sync_copy(xyzt_hbm.at[pl.ds((b * 3 + 0) * N, N)], xv)
    pltpu.sync_copy(xyzt_hbm.at[pl.ds((b * 3 + 1) * N, N)], yv)
    pltpu.sync_copy(xyzt_hbm.at[pl.ds((b * 3 + 2) * N, N)], zv)
    pltpu.sync_copy(psq_hbm.at[pl.ds(b * N, N)], psqv)
    pltpu.sync_copy(kpt_hbm.at[pl.ds((b * 3 + 0) * S + s0, KP_PER_WORKER)], kxv)
    pltpu.sync_copy(kpt_hbm.at[pl.ds((b * 3 + 1) * S + s0, KP_PER_WORKER)], kyv)
    pltpu.sync_copy(kpt_hbm.at[pl.ds((b * 3 + 2) * S + s0, KP_PER_WORKER)], kzv)
    pltpu.sync_copy(ksq_hbm.at[pl.ds(b * S + s0, KP_PER_WORKER)], ksqv)

    lanes16 = jnp.arange(16, dtype=jnp.int32)
    zeros16 = jnp.zeros((16,), jnp.int32)

    def scan_kp(j, idxg):
        jv = jnp.full((16,), j, jnp.int32)
        kxb = _bf16_round(plsc.load_gather(kxv, [jv]))
        kyb = _bf16_round(plsc.load_gather(kyv, [jv]))
        kzb = _bf16_round(plsc.load_gather(kzv, [jv]))
        ksq = plsc.load_gather(ksqv, [jv])

        UNROLL = 16

        def cond(carry):
            i, cntv = carry
            return jnp.logical_and(jnp.all(cntv < K), i < N // (16 * UNROLL))

        def body(carry):
            i, cntv = carry
            base = i * (16 * UNROLL)
            ms, cs, ps = [], [], []
            for u in range(UNROLL):
                ds = pl.ds(base + u * 16, 16)
                xc = _bf16_round(xv[ds])
                yc = _bf16_round(yv[ds])
                zc = _bf16_round(zv[ds])
                psqc = psqv[ds]
                dot = (kxb * xc + kyb * yc) + kzb * zc
                d = ((-2.0) * dot + ksq) + psqc
                m = d <= THR
                ms.append(m)
                cs.append(plsc.cumsum(m.astype(jnp.int32)))
                ps.append(plsc.all_reduce_population_count(m))
            off = cntv
            for u in range(UNROLL):
                pos = off + cs[u] - 1
                wm = jnp.logical_and(ms[u], pos < K)
                plsc.store_scatter(idxb, [pos],
                                   lanes16 + (base + u * 16), mask=wm)
                off = off + ps[u]
            return i + 1, off

        _, cntv = lax.while_loop(
            cond, body, (jnp.int32(0), jnp.zeros((16,), jnp.int32)))

        first = plsc.load_gather(idxb, [zeros16])
        fillv = jnp.where(cntv == 0, jnp.full((16,), N - 1, jnp.int32), first)
        for j2 in range(K // 16):
            lanes = lanes16 + 16 * j2
            cur = idxb[pl.ds(16 * j2, 16)]
            fin = jnp.where(lanes < cntv, cur, fillv)
            idxg[pl.ds(16 * j2, 16)] = fin + b * N

    def process_kp(j, idxg, rows, osem):
        jv = jnp.full((16,), j, jnp.int32)
        kx = plsc.load_gather(kxv, [jv])
        ky = plsc.load_gather(kyv, [jv])
        kz = plsc.load_gather(kzv, [jv])
        for j2 in range(K // 16):
            lanes = lanes16 + 16 * j2
            fin = idxg[pl.ds(16 * j2, 16)] - b * N
            gx = plsc.load_gather(xv, [fin])
            gy = plsc.load_gather(yv, [fin])
            gz = plsc.load_gather(zv, [fin])
            c0 = jnp.full((16,), 0, jnp.int32)
            git = plsc.load_gather(rows, [lanes, c0 + 68])
            plsc.store_scatter(rows, [lanes, c0], gx - kx)
            plsc.store_scatter(rows, [lanes, c0 + 1], gy - ky)
            plsc.store_scatter(rows, [lanes, c0 + 2], gz - kz)
            plsc.store_scatter(rows, [lanes, c0 + 3], git)
        kp = kp0 + j
        pltpu.async_copy(rows, out_hbm.at[pl.ds(kp * K, K)], osem)

    def wait_gather(idxg, rows, gsem):
        pltpu.make_async_copy(src_hbm.at[idxg], rows, gsem).wait()

    def wait_out(rows, osem):
        pltpu.make_async_copy(rows, out_hbm.at[pl.ds(0, K)], osem).wait()

    def pipelined(jj, carry):
        j0 = 2 * jj
        scan_kp(j0, idxg0)

        @pl.when(jj > 0)
        def _():
            wait_gather(idxg1, rows1, gsem1)
            process_kp(j0 - 1, idxg1, rows1, osem1)
            wait_out(rows0, osem0)

        pltpu.async_copy(src_hbm.at[idxg0], rows0, gsem0)

        scan_kp(j0 + 1, idxg1)
        wait_gather(idxg0, rows0, gsem0)
        process_kp(j0, idxg0, rows0, osem0)

        @pl.when(jj > 0)
        def _():
            wait_out(rows1, osem1)

        pltpu.async_copy(src_hbm.at[idxg1], rows1, gsem1)
        return carry

    lax.fori_loop(0, KP_PER_WORKER // 2, pipelined, 0)

    wait_gather(idxg1, rows1, gsem1)
    process_kp(KP_PER_WORKER - 1, idxg1, rows1, osem1)
    wait_out(rows0, osem0)
    wait_out(rows1, osem1)


@jax.jit
def kernel(source, source_intensity, keypoints):
    src_pad = jnp.concatenate(
        [jnp.zeros((B, N, 4), jnp.float32), source, source_intensity,
         jnp.zeros((B, N, 59), jnp.float32)], axis=-1).reshape(B * N, 128)
    xyz = source[:, :, :3]
    kpt3 = keypoints[:, :, :3]
    xyzt = jnp.transpose(xyz, (0, 2, 1)).reshape(B * 3 * N)
    kpt = jnp.transpose(kpt3, (0, 2, 1)).reshape(B * 3 * S)
    psq = jnp.sum(xyz ** 2, axis=-1).reshape(B * N)
    ksq = jnp.sum(kpt3 ** 2, axis=-1).reshape(B * S)

    mesh = plsc.VectorSubcoreMesh(
        core_axis_name="c", subcore_axis_name="s",
        num_cores=NUM_CORES, num_subcores=NUM_SUBCORES)
    call = pl.kernel(
        _ball_query_body,
        out_type=jax.ShapeDtypeStruct((B * S * K, 128), jnp.float32),
        mesh=mesh,
        compiler_params=pltpu.CompilerParams(needs_layout_passes=False),
        scratch_types=[
            pltpu.VMEM((N,), jnp.float32),
            pltpu.VMEM((N,), jnp.float32),
            pltpu.VMEM((N,), jnp.float32),
            pltpu.VMEM((N,), jnp.float32),
            pltpu.VMEM((KP_PER_WORKER,), jnp.float32),
            pltpu.VMEM((KP_PER_WORKER,), jnp.float32),
            pltpu.VMEM((KP_PER_WORKER,), jnp.float32),
            pltpu.VMEM((KP_PER_WORKER,), jnp.float32),
            pltpu.VMEM((2 * K,), jnp.int32),
            pltpu.VMEM((K,), jnp.int32),
            pltpu.VMEM((K,), jnp.int32),
            pltpu.VMEM((K, 128), jnp.float32),
            pltpu.VMEM((K, 128), jnp.float32),
            pltpu.SemaphoreType.DMA,
            pltpu.SemaphoreType.DMA,
            pltpu.SemaphoreType.DMA,
            pltpu.SemaphoreType.DMA,
        ],
    )
    out = call(src_pad, xyzt, psq, kpt, ksq)
    return out.reshape(B, S, K, 128)[..., :68]
